# baseline (speedup 1.0000x reference)
import functools

import jax
import jax.numpy as jnp
from jax import lax
from jax.experimental import pallas as pl
from jax.experimental.pallas import tpu as pltpu
from jax.experimental.pallas import tpu_sc as plsc

BATCH = 16384
EMBED = 64
NUM_LABELS = 16
NUM_WORKERS = 32
BPW = BATCH // NUM_WORKERS
CHUNK = 128
NCHUNK = BPW // CHUNK
LANES = 16

_mesh = plsc.VectorSubcoreMesh(core_axis_name="c", subcore_axis_name="s")


@functools.partial(
    pl.kernel,
    out_type=jax.ShapeDtypeStruct((BATCH,), jnp.float32),
    mesh=_mesh,
    compiler_params=pltpu.CompilerParams(needs_layout_passes=False,
                                         use_tc_tiling_on_sc=False),
    scratch_types=[
        pltpu.VMEM((NCHUNK, CHUNK), jnp.int32),
        pltpu.VMEM((BPW,), jnp.int32),
        pltpu.VMEM((BPW, EMBED), jnp.float32),
        pltpu.VMEM((NUM_LABELS * EMBED,), jnp.float32),
        pltpu.VMEM((BPW,), jnp.float32),
        pltpu.SemaphoreType.DMA,
    ],
)
def _pmi_dot(widx_hbm, lidx_hbm, wtab_hbm, ltab_hbm, out_hbm,
             idx_v, lbl_v, rows_v, ltab_v, out_v, sem):
    wid = lax.axis_index("s") * 2 + lax.axis_index("c")
    base = wid * BPW

    for c in range(NCHUNK):
        pltpu.sync_copy(widx_hbm.at[pl.ds(base + c * CHUNK, CHUNK)],
                        idx_v.at[c])
    pltpu.sync_copy(lidx_hbm.at[pl.ds(base, BPW)], lbl_v)
    pltpu.sync_copy(ltab_hbm, ltab_v)
    copies = [
        pltpu.async_copy(
            wtab_hbm.at[idx_v.at[c]],
            rows_v.at[pl.ds(c * CHUNK, CHUNK)],
            sem,
        )
        for c in range(NCHUNK)
    ]
    for cp in copies:
        cp.wait()

    iota = lax.iota(jnp.int32, LANES)

    def group(g, carry):
        gbase = pl.multiple_of(g * LANES, LANES)
        lbl_vec = lbl_v[pl.ds(gbase, LANES)]
        acc = jnp.zeros((LANES,), jnp.float32)
        for j in range(LANES):
            row = rows_v.at[gbase + j]
            off = lbl_vec[j] * EMBED
            p = row[pl.ds(0, LANES)] * ltab_v[pl.ds(off, LANES)]
            for c in range(1, EMBED // LANES):
                p = p + (row[pl.ds(c * LANES, LANES)]
                         * ltab_v[pl.ds(off + c * LANES, LANES)])
            acc = jnp.where(iota == j, jnp.sum(p), acc)
        out_v[pl.ds(gbase, LANES)] = acc
        return carry

    lax.fori_loop(0, BPW // LANES, group, 0)

    pltpu.sync_copy(out_v, out_hbm.at[pl.ds(base, BPW)])


def kernel(data, target, word_embedding, label_embedding):
    del target
    data = data.astype(jnp.int32)
    return _pmi_dot(data[:, 0], data[:, 1], word_embedding,
                    label_embedding.reshape(-1))
